# baseline (speedup 1.0000x reference)
import functools

import jax
import jax.numpy as jnp
import numpy as np
from jax import lax
from jax.experimental import pallas as pl
from jax.experimental.pallas import tpu as pltpu
from jax.experimental.pallas import tpu_sc as plsc

NC = 2
NS = 16
NW = NC * NS

CHUNK = 100
D = 64
PERIOD = 200


def _pos_table(seq_len, d_model):
    pos = jnp.arange(seq_len, dtype=jnp.float32)[:, None]
    div = jnp.exp(jnp.arange(0, d_model, 2, dtype=jnp.float32)
                  * -(np.log(10000.0) / d_model))
    pe = jnp.zeros((seq_len, d_model), dtype=jnp.float32)
    pe = pe.at[:, 0::2].set(jnp.sin(pos * div))
    pe = pe.at[:, 1::2].set(jnp.cos(pos * div))
    return pe


def _make_kernel(n_rows):
    per_w = n_rows // NW
    n_chunks = per_w // CHUNK
    mesh = plsc.VectorSubcoreMesh(core_axis_name="c", subcore_axis_name="s")

    @functools.partial(
        pl.kernel,
        mesh=mesh,
        compiler_params=pltpu.CompilerParams(use_tc_tiling_on_sc=False),
        out_type=jax.ShapeDtypeStruct((n_rows // CHUNK, CHUNK, D), jnp.float32),
        scratch_types=[
            pltpu.VMEM((n_chunks, CHUNK), jnp.int32),
            pltpu.VMEM((PERIOD, D), jnp.float32),
            pltpu.VMEM((4, CHUNK, D), jnp.float32),
            pltpu.SemaphoreType.DMA((4,)),
            pltpu.SemaphoreType.DMA((4,)),
        ],
    )
    def body(idx_hbm, table_hbm, pe_hbm, out_hbm, idx_v, pe_v, rows_v, gsem, ssem):
        wid = lax.axis_index("s") * NC + lax.axis_index("c")
        obase = wid * n_chunks

        pltpu.sync_copy(idx_hbm.at[wid], idx_v)
        pltpu.sync_copy(pe_hbm, pe_v)

        def fire_gather(j, b):
            pltpu.async_copy(table_hbm.at[idx_v.at[j]], rows_v.at[b], gsem.at[b])

        def wait_gather(j, b):
            pltpu.make_async_copy(
                table_hbm.at[idx_v.at[j]], rows_v.at[b], gsem.at[b]).wait()

        def fire_store(j, b):
            pltpu.async_copy(rows_v.at[b], out_hbm.at[obase + j], ssem.at[b])

        def wait_store(j, b):
            pltpu.make_async_copy(
                rows_v.at[b], out_hbm.at[obase + j], ssem.at[b]).wait()

        def add_pe(b, parity):
            pbase = parity * CHUNK

            def row_body(r, carry):
                for c in range(D // 16):
                    vec = pe_v[pbase + r, pl.ds(c * 16, 16)]
                    plsc.addupdate(rows_v.at[b, r, pl.ds(c * 16, 16)], vec)
                return carry

            lax.fori_loop(0, CHUNK, row_body, 0, unroll=4)

        fire_gather(0, 0)
        fire_gather(1, 1)

        wait_gather(0, 0)
        add_pe(0, 0)
        fire_store(0, 0)
        fire_gather(2, 2)

        wait_gather(1, 1)
        add_pe(1, 1)
        fire_store(1, 1)
        fire_gather(3, 3)

        def steady(jj, carry):
            j0 = 2 + jj * 4
            for b_off in range(4):
                j = j0 + b_off
                b = (2 + b_off) % 4
                parity = b_off % 2
                wait_gather(j, b)
                add_pe(b, parity)
                fire_store(j, b)
                wait_store(j - 2, (b + 2) % 4)
                fire_gather(j + 2, (b + 2) % 4)
            return carry

        lax.fori_loop(0, (n_chunks - 4) // 4, steady, 0)

        jt = n_chunks - 2
        wait_gather(jt, jt % 4)
        add_pe(jt % 4, jt % 2)
        fire_store(jt, jt % 4)
        wait_store(jt - 2, (jt - 2) % 4)

        jt = n_chunks - 1
        wait_gather(jt, jt % 4)
        add_pe(jt % 4, jt % 2)
        fire_store(jt, jt % 4)
        wait_store(jt - 2, (jt - 2) % 4)

        wait_store(n_chunks - 2, (n_chunks - 2) % 4)
        wait_store(n_chunks - 1, (n_chunks - 1) % 4)

    return body


def kernel(sequence, token_table):
    batch, seq_len = sequence.shape
    vocab, d_model = token_table.shape
    n_rows = batch * seq_len
    pe = _pos_table(seq_len, d_model)
    idx = sequence.reshape(NW, n_rows // NW // CHUNK, CHUNK).astype(jnp.int32)
    out = _make_kernel(n_rows)(idx, token_table, pe)
    return out.reshape(batch, seq_len, d_model)
